# baseline (speedup 1.0000x reference)
import functools

import jax
import jax.numpy as jnp
from jax.experimental import pallas as pl
from jax.experimental.pallas import tpu as pltpu


def _cdiv(a, b):
    return (a + b - 1) // b


def _masked_dot(a, t, col0, valid, k0):
    if k0 >= col0 + a.shape[1]:
        return jnp.dot(a, t, preferred_element_type=jnp.float32)
    s = k0 - col0
    a_head = a[:, :s]
    a_tail = a[:, s:]
    col = k0 + jax.lax.broadcasted_iota(jnp.int32, a_tail.shape, 1)
    a_tail = jnp.where(col < valid, a_tail, 0.0)
    acc = jnp.dot(a_head, t[:s, :], preferred_element_type=jnp.float32)
    acc += jnp.dot(a_tail, t[s:, :], preferred_element_type=jnp.float32)
    return acc


def _stage_a(nw, tm, kh, g2a_ref, g2b_ref, x_ref, p_ref, o_ref):
    gx = jnp.dot(g2a_ref[...], x_ref[:kh, :],
                 preferred_element_type=jnp.float32)
    gx += jnp.dot(g2b_ref[...], x_ref[kh:, :],
                  preferred_element_type=jnp.float32)
    acc = jnp.dot(gx, p_ref[...], preferred_element_type=jnp.float32)
    row = pl.program_id(0) * tm + jax.lax.broadcasted_iota(
        jnp.int32, acc.shape, 0)
    o_ref[...] = jnp.where(row < nw, acc, 0.0)


def _stage_b(nw, tm, kh, k0, wa_ref, wb_ref, t_ref, o_ref):
    acc = _masked_dot(wa_ref[...], t_ref[:kh, :], 0, nw, k0)
    acc += _masked_dot(wb_ref[...], t_ref[kh:, :], kh, nw, k0)
    row = pl.program_id(0) * tm + jax.lax.broadcasted_iota(
        jnp.int32, acc.shape, 0)
    o_ref[...] = jnp.where(row < nw, acc, 0.0)


def _stage_c(nw, kh, k0, g1a_ref, g1b_ref, t_ref, b_ref, o_ref):
    acc = _masked_dot(g1a_ref[...], t_ref[:kh, :], 0, nw, k0)
    acc += _masked_dot(g1b_ref[...], t_ref[kh:, :], kh, nw, k0)
    o_ref[...] = acc + b_ref[...]


def kernel(g1, g2, x, weight, p, bias):
    m, nw = g1.shape
    in_dim = x.shape[1]
    out_dim = p.shape[1]

    tm = 512
    nwp = _cdiv(nw, tm) * tm
    k0 = (nw // 256) * 256
    khm = m // 2
    khw = nwp // 2

    parallel = pltpu.CompilerParams(dimension_semantics=("parallel",))

    def half_spec(width, half):
        return pl.BlockSpec((tm, width), lambda i, h=half: (i, h))

    def resident(shape):
        return pl.BlockSpec(shape, lambda i: (0, 0))

    t1 = pl.pallas_call(
        functools.partial(_stage_a, nw, tm, khm),
        out_shape=jax.ShapeDtypeStruct((nwp, out_dim), jnp.float32),
        grid=(nwp // tm,),
        in_specs=[
            half_spec(khm, 0),
            half_spec(khm, 1),
            resident((m, in_dim)),
            resident((in_dim, out_dim)),
        ],
        out_specs=pl.BlockSpec((tm, out_dim), lambda i: (i, 0)),
        compiler_params=parallel,
    )(g2, g2, x, p)

    t2 = pl.pallas_call(
        functools.partial(_stage_b, nw, tm, khw, k0),
        out_shape=jax.ShapeDtypeStruct((nwp, out_dim), jnp.float32),
        grid=(nwp // tm,),
        in_specs=[
            half_spec(khw, 0),
            half_spec(khw, 1),
            resident((nwp, out_dim)),
        ],
        out_specs=pl.BlockSpec((tm, out_dim), lambda i: (i, 0)),
        compiler_params=parallel,
    )(weight, weight, t1)

    out = pl.pallas_call(
        functools.partial(_stage_c, nw, khw, k0),
        out_shape=jax.ShapeDtypeStruct((m, out_dim), jnp.float32),
        grid=(m // tm,),
        in_specs=[
            half_spec(khw, 0),
            half_spec(khw, 1),
            resident((nwp, out_dim)),
            resident((1, out_dim)),
        ],
        out_specs=pl.BlockSpec((tm, out_dim), lambda i: (i, 0)),
        compiler_params=parallel,
    )(g1, g1, t2, bias.reshape(1, out_dim))

    return out
